# baseline (speedup 1.0000x reference)
import functools

import jax
import jax.numpy as jnp
from jax import lax
from jax.experimental import pallas as pl
from jax.experimental.pallas import tpu as pltpu
from jax.experimental.pallas import tpu_sc as plsc

BATCH = 4
SEQ = 8192
DIM = 1024
CHUNK = 64
SH_CHUNK = 512


def _pos_embed_kernel(table_hbm, out_hbm, buf, shared, wsem):
    info = plsc.get_sparse_core_info()
    nc, ns = info.num_cores, info.num_subcores
    nw = nc * ns
    rows_per_w = SEQ // nw
    cid = lax.axis_index("c")
    sid = lax.axis_index("s")
    wid = sid * nc + cid
    base = wid * rows_per_w

    for i in range(rows_per_w // CHUNK):
        row = base + i * CHUNK
        pltpu.sync_copy(table_hbm.at[pl.ds(row, CHUNK)], buf)
        handles = [
            pltpu.async_copy(buf, out_hbm.at[b, pl.ds(row, CHUNK)], wsem)
            for b in range(2)
        ]
        for h in handles:
            h.wait()

    rows_per_core = SEQ // nc
    cbase = cid * rows_per_core

    @pl.when(sid == 0)
    def _():
        for i in range(rows_per_core // SH_CHUNK):
            row = cbase + i * SH_CHUNK
            pltpu.sync_copy(table_hbm.at[pl.ds(row, SH_CHUNK)], shared)
            for b in range(2, BATCH):
                pltpu.sync_copy(shared, out_hbm.at[b, pl.ds(row, SH_CHUNK)])


@jax.jit
def _pos_embed(table):
    mesh = plsc.VectorSubcoreMesh(core_axis_name="c", subcore_axis_name="s")
    fn = functools.partial(
        pl.kernel,
        mesh=mesh,
        out_type=jax.ShapeDtypeStruct((BATCH, SEQ, DIM), jnp.float32),
        scratch_types=[
            pltpu.VMEM((CHUNK, DIM), jnp.float32),
            pltpu.VMEM_SHARED((SH_CHUNK, DIM), jnp.float32),
            pltpu.SemaphoreType.DMA,
        ],
    )(_pos_embed_kernel)
    return fn(table)


def kernel(inputs, table):
    del inputs
    return _pos_embed(table)


# speedup vs baseline: 1.7432x; 1.7432x over previous
import functools

import jax
import jax.numpy as jnp
from jax import lax
from jax.experimental import pallas as pl
from jax.experimental.pallas import tpu as pltpu
from jax.experimental.pallas import tpu_sc as plsc

BATCH = 4
SEQ = 8192
DIM = 1024
CHUNK = 56


def _pos_embed_kernel(table_hbm, out_hbm, buf0, buf1, rs0, rs1, ws0, ws1):
    info = plsc.get_sparse_core_info()
    nc, ns = info.num_cores, info.num_subcores
    nw = nc * ns
    rows_per_w = SEQ // nw
    wid = lax.axis_index("s") * nc + lax.axis_index("c")
    base = wid * rows_per_w

    sizes = [CHUNK] * (rows_per_w // CHUNK)
    if rows_per_w % CHUNK:
        sizes.append(rows_per_w % CHUNK)
    offs = [sum(sizes[:i]) for i in range(len(sizes))]
    n = len(sizes)

    bufs = (buf0, buf1)
    rsems = (rs0, rs1)
    wsems = (ws0, ws1)

    def read(i):
        return pltpu.async_copy(
            table_hbm.at[pl.ds(base + offs[i], sizes[i])],
            bufs[i % 2].at[pl.ds(0, sizes[i])], rsems[i % 2])

    def write(i, b):
        return pltpu.async_copy(
            bufs[i % 2].at[pl.ds(0, sizes[i])],
            out_hbm.at[b, pl.ds(base + offs[i], sizes[i])], wsems[i % 2])

    rd = [None] * n
    wr = [None] * n
    rd[0] = read(0)
    if n > 1:
        rd[1] = read(1)
    for i in range(n):
        rd[i].wait()
        wr[i] = [write(i, b) for b in range(BATCH)]
        if i + 2 < n:
            for h in wr[i]:
                h.wait()
            rd[i + 2] = read(i + 2)
    for i in range(max(0, n - 2), n):
        for h in wr[i]:
            h.wait()


@jax.jit
def _pos_embed(table):
    mesh = plsc.VectorSubcoreMesh(core_axis_name="c", subcore_axis_name="s")
    fn = functools.partial(
        pl.kernel,
        mesh=mesh,
        out_type=jax.ShapeDtypeStruct((BATCH, SEQ, DIM), jnp.float32),
        scratch_types=[
            pltpu.VMEM((CHUNK, DIM), jnp.float32),
            pltpu.VMEM((CHUNK, DIM), jnp.float32),
            pltpu.SemaphoreType.DMA,
            pltpu.SemaphoreType.DMA,
            pltpu.SemaphoreType.DMA,
            pltpu.SemaphoreType.DMA,
        ],
    )(_pos_embed_kernel)
    return fn(table)


def kernel(inputs, table):
    del inputs
    return _pos_embed(table)


# speedup vs baseline: 2.4160x; 1.3860x over previous
import jax
import jax.numpy as jnp
from jax.experimental import pallas as pl

BATCH = 4
SEQ = 8192
DIM = 1024
TC_BLOCK = 512


def _tc_body(table_ref, out_ref):
    rows = table_ref[...]
    for b in range(BATCH):
        out_ref[b] = rows


@jax.jit
def _pos_embed(table):
    return pl.pallas_call(
        _tc_body,
        grid=(SEQ // TC_BLOCK,),
        in_specs=[pl.BlockSpec((TC_BLOCK, DIM), lambda i: (i, 0))],
        out_specs=pl.BlockSpec((BATCH, TC_BLOCK, DIM), lambda i: (0, i, 0)),
        out_shape=jax.ShapeDtypeStruct((BATCH, SEQ, DIM), jnp.float32),
    )(table)


def kernel(inputs, table):
    del inputs
    return _pos_embed(table)
